# Initial kernel scaffold; baseline (speedup 1.0000x reference)
import jax
import jax.numpy as jnp
from jax.experimental import pallas as pl


def kernel(pos_src, pos_dst, neg_src, neg_dst, node_embeddings):
    raise NotImplementedError("write your pallas kernel here")


# speedup vs baseline: 1.2240x; 1.2240x over previous
import functools

import jax
import jax.numpy as jnp
from jax import lax
from jax.experimental import pallas as pl
from jax.experimental.pallas import tpu as pltpu
from jax.experimental.pallas import tpu_sc as plsc

N_NODES = 10000
D = 128
E_HALF = 320000
E = 2 * E_HALF

NC = 2
NS = 16
L = 16
NW = NC * NS
EPW = E // NW
B = 400
NCHUNK = EPW // B
GROUPS = B // L

_mesh = plsc.VectorSubcoreMesh(core_axis_name="c", subcore_axis_name="s")


@functools.partial(
    pl.kernel,
    mesh=_mesh,
    out_type=jax.ShapeDtypeStruct((E,), jnp.float32),
    scratch_types=[
        pltpu.VMEM((B,), jnp.int32),
        pltpu.VMEM((B,), jnp.int32),
        pltpu.VMEM((B, D), jnp.float32),
        pltpu.VMEM((B, D), jnp.float32),
        pltpu.VMEM((B,), jnp.float32),
        pltpu.SemaphoreType.DMA,
    ],
    compiler_params=pltpu.CompilerParams(needs_layout_passes=False),
)
def _scores_sc(src_hbm, dst_hbm, emb_hbm, out_hbm,
               sidx_v, didx_v, srow_v, drow_v, sc_v, sem):
    wid = lax.axis_index("s") * NC + lax.axis_index("c")
    sgn = jnp.where(wid < NW // 2, -1.0, 1.0).astype(jnp.float32)
    base = wid * EPW

    def chunk_body(k, carry):
        off = base + k * B
        pltpu.sync_copy(src_hbm.at[pl.ds(off, B)], sidx_v)
        pltpu.sync_copy(dst_hbm.at[pl.ds(off, B)], didx_v)
        cp1 = pltpu.async_copy(emb_hbm.at[sidx_v], srow_v, sem)
        cp2 = pltpu.async_copy(emb_hbm.at[didx_v], drow_v, sem)
        cp1.wait()
        cp2.wait()

        def group_body(g, c2):
            erow = g * L + lax.iota(jnp.int32, L)
            acc = jnp.zeros((L,), jnp.float32)
            for j in range(D):
                cj = jnp.full((L,), j, jnp.int32)
                a = plsc.load_gather(srow_v, [erow, cj])
                b = plsc.load_gather(drow_v, [erow, cj])
                acc = acc + a * b
            sc_v[pl.ds(g * L, L)] = acc * sgn
            return c2

        lax.fori_loop(0, GROUPS, group_body, 0)
        pltpu.sync_copy(sc_v, out_hbm.at[pl.ds(off, B)])
        return carry

    lax.fori_loop(0, NCHUNK, chunk_body, 0)


def _loss_body(x_ref, o_ref):
    v = x_ref[...]
    sp = jnp.maximum(v, 0.0) + jnp.log1p(jnp.exp(-jnp.abs(v)))
    o_ref[0, 0] = jnp.sum(sp) * (1.0 / E_HALF)


def _loss_tc(scores):
    x = scores.reshape(E // 128, 128)
    out = pl.pallas_call(
        _loss_body,
        out_shape=jax.ShapeDtypeStruct((1, 1), jnp.float32),
        out_specs=pl.BlockSpec(memory_space=pltpu.SMEM),
    )(x)
    return out[0, 0]


def kernel(pos_src, pos_dst, neg_src, neg_dst, node_embeddings):
    src = jnp.concatenate([pos_src, neg_src]).astype(jnp.int32)
    dst = jnp.concatenate([pos_dst, neg_dst]).astype(jnp.int32)
    scores = _scores_sc(src, dst, node_embeddings)
    return _loss_tc(scores)


# speedup vs baseline: 6.7648x; 5.5268x over previous
import functools

import jax
import jax.numpy as jnp
from jax import lax
from jax.experimental import pallas as pl
from jax.experimental.pallas import tpu as pltpu
from jax.experimental.pallas import tpu_sc as plsc

N_NODES = 10000
D = 128
E_HALF = 320000
E = 2 * E_HALF

NC = 2
NS = 16
L = 16
NW = NC * NS
EPW = E // NW
B = 400
NCHUNK = EPW // B
GROUPS = B // L

_mesh = plsc.VectorSubcoreMesh(core_axis_name="c", subcore_axis_name="s")


@functools.partial(
    pl.kernel,
    mesh=_mesh,
    out_type=jax.ShapeDtypeStruct((E,), jnp.float32),
    scratch_types=[
        pltpu.VMEM((B,), jnp.int32),
        pltpu.VMEM((B,), jnp.int32),
        pltpu.VMEM((B, D), jnp.float32),
        pltpu.VMEM((B, D), jnp.float32),
        pltpu.VMEM((B,), jnp.float32),
        pltpu.SemaphoreType.DMA,
    ],
    compiler_params=pltpu.CompilerParams(needs_layout_passes=False),
)
def _scores_sc(src_hbm, dst_hbm, emb_hbm, out_hbm,
               sidx_v, didx_v, srow_v, drow_v, sc_v, sem):
    wid = lax.axis_index("s") * NC + lax.axis_index("c")
    sgn = jnp.where(wid < NW // 2, -1.0, 1.0).astype(jnp.float32)
    base = wid * EPW

    def chunk_body(k, carry):
        off = base + k * B
        pltpu.sync_copy(src_hbm.at[pl.ds(off, B)], sidx_v)
        pltpu.sync_copy(dst_hbm.at[pl.ds(off, B)], didx_v)
        cp1 = pltpu.async_copy(emb_hbm.at[sidx_v], srow_v, sem)
        cp2 = pltpu.async_copy(emb_hbm.at[didx_v], drow_v, sem)
        cp1.wait()
        cp2.wait()

        lane = lax.iota(jnp.int32, L)
        last = lane == (L - 1)

        @plsc.parallel_loop(0, B, 1, unroll=4)
        def edge_body(e):
            acc = jnp.zeros((L,), jnp.float32)
            for j in range(D // L):
                a = srow_v[e, pl.ds(j * L, L)]
                b = drow_v[e, pl.ds(j * L, L)]
                acc = acc + a * b
            tot = plsc.cumsum(acc) * sgn
            plsc.store_scatter(sc_v, [jnp.full((L,), e, jnp.int32)],
                               tot, mask=last)
        pltpu.sync_copy(sc_v, out_hbm.at[pl.ds(off, B)])
        return carry

    lax.fori_loop(0, NCHUNK, chunk_body, 0)


def _loss_body(x_ref, o_ref):
    v = x_ref[...]
    sp = jnp.maximum(v, 0.0) + jnp.log1p(jnp.exp(-jnp.abs(v)))
    o_ref[0, 0] = jnp.sum(sp) * (1.0 / E_HALF)


def _loss_tc(scores):
    x = scores.reshape(E // 128, 128)
    out = pl.pallas_call(
        _loss_body,
        out_shape=jax.ShapeDtypeStruct((1, 1), jnp.float32),
        out_specs=pl.BlockSpec(memory_space=pltpu.SMEM),
    )(x)
    return out[0, 0]


def kernel(pos_src, pos_dst, neg_src, neg_dst, node_embeddings):
    src = jnp.concatenate([pos_src, neg_src]).astype(jnp.int32)
    dst = jnp.concatenate([pos_dst, neg_dst]).astype(jnp.int32)
    scores = _scores_sc(src, dst, node_embeddings)
    return _loss_tc(scores)


# speedup vs baseline: 9.9080x; 1.4646x over previous
import functools

import jax
import jax.numpy as jnp
from jax import lax
from jax.experimental import pallas as pl
from jax.experimental.pallas import tpu as pltpu
from jax.experimental.pallas import tpu_sc as plsc

N_NODES = 10000
D = 128
E_HALF = 320000
E = 2 * E_HALF

NC = 2
NS = 16
L = 16
NW = NC * NS
EPW = E // NW
B = 200
NCHUNK = EPW // B
HALF_ITERS = NCHUNK // 2

_mesh = plsc.VectorSubcoreMesh(core_axis_name="c", subcore_axis_name="s")


@functools.partial(
    pl.kernel,
    mesh=_mesh,
    out_type=jax.ShapeDtypeStruct((E,), jnp.float32),
    scratch_types=[
        [pltpu.VMEM((B,), jnp.int32)] * 2,
        [pltpu.VMEM((B,), jnp.int32)] * 2,
        [pltpu.VMEM((B, D), jnp.float32)] * 2,
        [pltpu.VMEM((B, D), jnp.float32)] * 2,
        [pltpu.VMEM((B,), jnp.float32)] * 2,
        [pltpu.SemaphoreType.DMA] * 2,
    ],
    compiler_params=pltpu.CompilerParams(needs_layout_passes=False),
)
def _scores_sc(src_hbm, dst_hbm, emb_hbm, out_hbm,
               sidx, didx, srow, drow, sc, sem):
    wid = lax.axis_index("s") * NC + lax.axis_index("c")
    sgn = jnp.where(wid < NW // 2, -1.0, 1.0).astype(jnp.float32)
    base = wid * EPW
    lane = lax.iota(jnp.int32, L)
    last = lane == (L - 1)

    def start_fetch(k, b):
        off = base + k * B
        pltpu.sync_copy(src_hbm.at[pl.ds(off, B)], sidx[b])
        pltpu.sync_copy(dst_hbm.at[pl.ds(off, B)], didx[b])
        pltpu.async_copy(emb_hbm.at[sidx[b]], srow[b], sem[b])
        pltpu.async_copy(emb_hbm.at[didx[b]], drow[b], sem[b])

    def compute(k, b):
        pltpu.make_async_copy(emb_hbm.at[sidx[b]], srow[b], sem[b]).wait()
        pltpu.make_async_copy(emb_hbm.at[didx[b]], drow[b], sem[b]).wait()
        srow_v, drow_v, sc_v = srow[b], drow[b], sc[b]

        @plsc.parallel_loop(0, B, 1, unroll=4)
        def edge_body(e):
            acc = jnp.zeros((L,), jnp.float32)
            for j in range(D // L):
                a = srow_v[e, pl.ds(j * L, L)]
                b_ = drow_v[e, pl.ds(j * L, L)]
                acc = acc + a * b_
            tot = plsc.cumsum(acc) * sgn
            plsc.store_scatter(sc_v, [jnp.full((L,), e, jnp.int32)],
                               tot, mask=last)
        pltpu.sync_copy(sc_v, out_hbm.at[pl.ds(base + k * B, B)])

    start_fetch(0, 0)

    def body(g, carry):
        start_fetch(2 * g + 1, 1)
        compute(2 * g, 0)

        @pl.when(g < HALF_ITERS - 1)
        def _():
            start_fetch(2 * g + 2, 0)

        compute(2 * g + 1, 1)
        return carry

    lax.fori_loop(0, HALF_ITERS, body, 0)


def _loss_body(x_ref, o_ref):
    v = x_ref[...]
    sp = jnp.maximum(v, 0.0) + jnp.log1p(jnp.exp(-jnp.abs(v)))
    o_ref[0, 0] = jnp.sum(sp) * (1.0 / E_HALF)


def _loss_tc(scores):
    x = scores.reshape(E // 128, 128)
    out = pl.pallas_call(
        _loss_body,
        out_shape=jax.ShapeDtypeStruct((1, 1), jnp.float32),
        out_specs=pl.BlockSpec(memory_space=pltpu.SMEM),
    )(x)
    return out[0, 0]


def kernel(pos_src, pos_dst, neg_src, neg_dst, node_embeddings):
    src = jnp.concatenate([pos_src, neg_src]).astype(jnp.int32)
    dst = jnp.concatenate([pos_dst, neg_dst]).astype(jnp.int32)
    scores = _scores_sc(src, dst, node_embeddings)
    return _loss_tc(scores)


# speedup vs baseline: 11.1146x; 1.1218x over previous
import functools

import jax
import jax.numpy as jnp
from jax import lax
from jax.experimental import pallas as pl
from jax.experimental.pallas import tpu as pltpu
from jax.experimental.pallas import tpu_sc as plsc

N_NODES = 10000
D = 128
E_HALF = 320000
E = 2 * E_HALF

NC = 2
NS = 16
L = 16
NW = NC * NS
EPW = E // NW
B = 200
NCHUNK = EPW // B
HALF_ITERS = NCHUNK // 2

_mesh = plsc.VectorSubcoreMesh(core_axis_name="c", subcore_axis_name="s")


@functools.partial(
    pl.kernel,
    mesh=_mesh,
    out_type=jax.ShapeDtypeStruct((E,), jnp.float32),
    scratch_types=[
        [pltpu.VMEM((B,), jnp.int32)] * 2,
        [pltpu.VMEM((B,), jnp.int32)] * 2,
        [pltpu.VMEM((B, D // 2), jnp.int32)] * 2,
        [pltpu.VMEM((B, D // 2), jnp.int32)] * 2,
        [pltpu.VMEM((B,), jnp.float32)] * 2,
        [pltpu.SemaphoreType.DMA] * 2,
    ],
    compiler_params=pltpu.CompilerParams(
        needs_layout_passes=False, use_tc_tiling_on_sc=False),
)
def _scores_sc(src_hbm, dst_hbm, emb_hbm, out_hbm,
               sidx, didx, srow, drow, sc, sem):
    wid = lax.axis_index("s") * NC + lax.axis_index("c")
    sgn = jnp.where(wid < NW // 2, -1.0, 1.0).astype(jnp.float32)
    base = wid * EPW
    lane = lax.iota(jnp.int32, L)
    last = lane == (L - 1)

    def start_fetch(k, b):
        off = base + k * B
        pltpu.sync_copy(src_hbm.at[pl.ds(off, B)], sidx[b])
        pltpu.sync_copy(dst_hbm.at[pl.ds(off, B)], didx[b])
        pltpu.async_copy(emb_hbm.at[sidx[b]], srow[b], sem[b])
        pltpu.async_copy(emb_hbm.at[didx[b]], drow[b], sem[b])

    def compute(k, b):
        pltpu.make_async_copy(emb_hbm.at[sidx[b]], srow[b], sem[b]).wait()
        pltpu.make_async_copy(emb_hbm.at[didx[b]], drow[b], sem[b]).wait()
        srow_v, drow_v, sc_v = srow[b], drow[b], sc[b]

        @plsc.parallel_loop(0, B, 1, unroll=4)
        def edge_body(e):
            acc = jnp.zeros((L,), jnp.float32)
            for j in range(D // (2 * L)):
                ai = srow_v[e, pl.ds(j * L, L)]
                bi = drow_v[e, pl.ds(j * L, L)]
                a2 = plsc.bitcast(ai, jnp.bfloat16)
                b2 = plsc.bitcast(bi, jnp.bfloat16)
                a0, a1 = plsc.unpack(a2, format=plsc.PackFormat.INTERLEAVED)
                b0, b1 = plsc.unpack(b2, format=plsc.PackFormat.INTERLEAVED)
                acc = acc + a0 * b0 + a1 * b1
            tot = plsc.cumsum(acc) * sgn
            plsc.store_scatter(sc_v, [jnp.full((L,), e, jnp.int32)],
                               tot, mask=last)
        pltpu.sync_copy(sc_v, out_hbm.at[pl.ds(base + k * B, B)])

    start_fetch(0, 0)

    def body(g, carry):
        start_fetch(2 * g + 1, 1)
        compute(2 * g, 0)

        @pl.when(g < HALF_ITERS - 1)
        def _():
            start_fetch(2 * g + 2, 0)

        compute(2 * g + 1, 1)
        return carry

    lax.fori_loop(0, HALF_ITERS, body, 0)


def _loss_body(x_ref, o_ref):
    v = x_ref[...]
    sp = jnp.maximum(v, 0.0) + jnp.log1p(jnp.exp(-jnp.abs(v)))
    o_ref[0, 0] = jnp.sum(sp) * (1.0 / E_HALF)


def _loss_tc(scores):
    x = scores.reshape(E // 128, 128)
    out = pl.pallas_call(
        _loss_body,
        out_shape=jax.ShapeDtypeStruct((1, 1), jnp.float32),
        out_specs=pl.BlockSpec(memory_space=pltpu.SMEM),
    )(x)
    return out[0, 0]


def kernel(pos_src, pos_dst, neg_src, neg_dst, node_embeddings):
    src = jnp.concatenate([pos_src, neg_src]).astype(jnp.int32)
    dst = jnp.concatenate([pos_dst, neg_dst]).astype(jnp.int32)
    emb_bf = node_embeddings.astype(jnp.bfloat16)
    emb_i32 = jax.lax.bitcast_convert_type(
        emb_bf.reshape(N_NODES, D // 2, 2), jnp.int32)
    scores = _scores_sc(src, dst, emb_i32)
    return _loss_tc(scores)


# speedup vs baseline: 14.9676x; 1.3467x over previous
import functools

import jax
import jax.numpy as jnp
from jax import lax
from jax.experimental import pallas as pl
from jax.experimental.pallas import tpu as pltpu
from jax.experimental.pallas import tpu_sc as plsc

N_NODES = 10000
D = 128
E_HALF = 320000
E = 2 * E_HALF

NC = 2
NS = 16
L = 16
NW = NC * NS
EPW = E // NW
B = 200
NCHUNK = EPW // B
HALF_ITERS = NCHUNK // 2

_mesh = plsc.VectorSubcoreMesh(core_axis_name="c", subcore_axis_name="s")


@functools.partial(
    pl.kernel,
    mesh=_mesh,
    out_type=jax.ShapeDtypeStruct((E,), jnp.float32),
    scratch_types=[
        pltpu.VMEM((EPW,), jnp.int32),
        pltpu.VMEM((EPW,), jnp.int32),
        [pltpu.VMEM((B, D // 2), jnp.int32)] * 2,
        [pltpu.VMEM((B, D // 2), jnp.int32)] * 2,
        [pltpu.VMEM((B,), jnp.float32)] * 2,
        [pltpu.SemaphoreType.DMA] * 2,
    ],
    compiler_params=pltpu.CompilerParams(
        needs_layout_passes=False, use_tc_tiling_on_sc=False),
)
def _scores_sc(src_hbm, dst_hbm, emb_hbm, out_hbm,
               sidx, didx, srow, drow, sc, sem):
    wid = lax.axis_index("s") * NC + lax.axis_index("c")
    sgn = jnp.where(wid < NW // 2, -1.0, 1.0).astype(jnp.float32)
    base = wid * EPW
    lane = lax.iota(jnp.int32, L)
    last = lane == (L - 1)

    pltpu.sync_copy(src_hbm.at[pl.ds(base, EPW)], sidx)
    pltpu.sync_copy(dst_hbm.at[pl.ds(base, EPW)], didx)

    def start_fetch(k, b):
        pltpu.async_copy(emb_hbm.at[sidx.at[pl.ds(k * B, B)]], srow[b], sem[b])
        pltpu.async_copy(emb_hbm.at[didx.at[pl.ds(k * B, B)]], drow[b], sem[b])

    def compute(k, b):
        pltpu.make_async_copy(
            emb_hbm.at[sidx.at[pl.ds(k * B, B)]], srow[b], sem[b]).wait()
        pltpu.make_async_copy(
            emb_hbm.at[didx.at[pl.ds(k * B, B)]], drow[b], sem[b]).wait()
        srow_v, drow_v, sc_v = srow[b], drow[b], sc[b]

        @plsc.parallel_loop(0, B, 1, unroll=4)
        def edge_body(e):
            acc = jnp.zeros((L,), jnp.float32)
            for j in range(D // (2 * L)):
                ai = srow_v[e, pl.ds(j * L, L)]
                bi = drow_v[e, pl.ds(j * L, L)]
                a2 = plsc.bitcast(ai, jnp.bfloat16)
                b2 = plsc.bitcast(bi, jnp.bfloat16)
                a0, a1 = plsc.unpack(a2, format=plsc.PackFormat.INTERLEAVED)
                b0, b1 = plsc.unpack(b2, format=plsc.PackFormat.INTERLEAVED)
                acc = acc + a0 * b0 + a1 * b1
            tot = plsc.cumsum(acc) * sgn
            plsc.store_scatter(sc_v, [jnp.full((L,), e, jnp.int32)],
                               tot, mask=last)
        pltpu.sync_copy(sc_v, out_hbm.at[pl.ds(base + k * B, B)])

    start_fetch(0, 0)

    def body(g, carry):
        start_fetch(2 * g + 1, 1)
        compute(2 * g, 0)

        @pl.when(g < HALF_ITERS - 1)
        def _():
            start_fetch(2 * g + 2, 0)

        compute(2 * g + 1, 1)
        return carry

    lax.fori_loop(0, HALF_ITERS, body, 0)


def _loss_body(x_ref, o_ref):
    v = x_ref[...]
    sp = jnp.maximum(v, 0.0) + jnp.log1p(jnp.exp(-jnp.abs(v)))
    o_ref[0, 0] = jnp.sum(sp) * (1.0 / E_HALF)


def _loss_tc(scores):
    x = scores.reshape(E // 128, 128)
    out = pl.pallas_call(
        _loss_body,
        out_shape=jax.ShapeDtypeStruct((1, 1), jnp.float32),
        out_specs=pl.BlockSpec(memory_space=pltpu.SMEM),
    )(x)
    return out[0, 0]


def kernel(pos_src, pos_dst, neg_src, neg_dst, node_embeddings):
    src = jnp.concatenate([pos_src, neg_src]).astype(jnp.int32)
    dst = jnp.concatenate([pos_dst, neg_dst]).astype(jnp.int32)
    emb_bf = node_embeddings.astype(jnp.bfloat16)
    emb_i32 = jax.lax.bitcast_convert_type(
        emb_bf.reshape(N_NODES, D // 2, 2), jnp.int32)
    scores = _scores_sc(src, dst, emb_i32)
    return _loss_tc(scores)


# speedup vs baseline: 15.2836x; 1.0211x over previous
import functools

import jax
import jax.numpy as jnp
from jax import lax
from jax.experimental import pallas as pl
from jax.experimental.pallas import tpu as pltpu
from jax.experimental.pallas import tpu_sc as plsc

N_NODES = 10000
D = 128
E_HALF = 320000
E = 2 * E_HALF

NC = 2
NS = 16
L = 16
NW = NC * NS
EPW = E // NW
B = 80
NCHUNK = EPW // B
HALF_ITERS = NCHUNK // 2

_mesh = plsc.VectorSubcoreMesh(core_axis_name="c", subcore_axis_name="s")


@functools.partial(
    pl.kernel,
    mesh=_mesh,
    out_type=jax.ShapeDtypeStruct((E,), jnp.float32),
    scratch_types=[
        pltpu.VMEM((EPW,), jnp.int32),
        pltpu.VMEM((EPW,), jnp.int32),
        [pltpu.VMEM((B, D // 2), jnp.int32)] * 2,
        [pltpu.VMEM((B, D // 2), jnp.int32)] * 2,
        [pltpu.VMEM((B,), jnp.float32)] * 2,
        [pltpu.SemaphoreType.DMA] * 2,
        pltpu.VMEM_SHARED((N_NODES, D // 2), jnp.int32),
    ],
    compiler_params=pltpu.CompilerParams(
        needs_layout_passes=False, use_tc_tiling_on_sc=False),
)
def _scores_sc(src_hbm, dst_hbm, emb_hbm, out_hbm,
               sidx, didx, srow, drow, sc, sem, emb_sh):
    wid = lax.axis_index("s") * NC + lax.axis_index("c")

    @pl.when(lax.axis_index("s") == 0)
    def _():
        pltpu.sync_copy(emb_hbm, emb_sh)

    sgn = jnp.where(wid < NW // 2, -1.0, 1.0).astype(jnp.float32)
    base = wid * EPW
    lane = lax.iota(jnp.int32, L)
    last = lane == (L - 1)

    pltpu.sync_copy(src_hbm.at[pl.ds(base, EPW)], sidx)
    pltpu.sync_copy(dst_hbm.at[pl.ds(base, EPW)], didx)
    plsc.subcore_barrier()

    def start_fetch(k, b):
        pltpu.async_copy(emb_sh.at[sidx.at[pl.ds(k * B, B)]], srow[b], sem[b])
        pltpu.async_copy(emb_sh.at[didx.at[pl.ds(k * B, B)]], drow[b], sem[b])

    def compute(k, b):
        pltpu.make_async_copy(
            emb_sh.at[sidx.at[pl.ds(k * B, B)]], srow[b], sem[b]).wait()
        pltpu.make_async_copy(
            emb_sh.at[didx.at[pl.ds(k * B, B)]], drow[b], sem[b]).wait()
        srow_v, drow_v, sc_v = srow[b], drow[b], sc[b]

        @plsc.parallel_loop(0, B, 1, unroll=4)
        def edge_body(e):
            acc = jnp.zeros((L,), jnp.float32)
            for j in range(D // (2 * L)):
                ai = srow_v[e, pl.ds(j * L, L)]
                bi = drow_v[e, pl.ds(j * L, L)]
                a2 = plsc.bitcast(ai, jnp.bfloat16)
                b2 = plsc.bitcast(bi, jnp.bfloat16)
                a0, a1 = plsc.unpack(a2, format=plsc.PackFormat.INTERLEAVED)
                b0, b1 = plsc.unpack(b2, format=plsc.PackFormat.INTERLEAVED)
                acc = acc + a0 * b0 + a1 * b1
            tot = plsc.cumsum(acc) * sgn
            plsc.store_scatter(sc_v, [jnp.full((L,), e, jnp.int32)],
                               tot, mask=last)
        pltpu.sync_copy(sc_v, out_hbm.at[pl.ds(base + k * B, B)])

    start_fetch(0, 0)

    def body(g, carry):
        start_fetch(2 * g + 1, 1)
        compute(2 * g, 0)

        @pl.when(g < HALF_ITERS - 1)
        def _():
            start_fetch(2 * g + 2, 0)

        compute(2 * g + 1, 1)
        return carry

    lax.fori_loop(0, HALF_ITERS, body, 0)


def _loss_body(x_ref, o_ref):
    v = x_ref[...]
    sp = jnp.maximum(v, 0.0) + jnp.log1p(jnp.exp(-jnp.abs(v)))
    o_ref[0, 0] = jnp.sum(sp) * (1.0 / E_HALF)


def _loss_tc(scores):
    x = scores.reshape(E // 128, 128)
    out = pl.pallas_call(
        _loss_body,
        out_shape=jax.ShapeDtypeStruct((1, 1), jnp.float32),
        out_specs=pl.BlockSpec(memory_space=pltpu.SMEM),
    )(x)
    return out[0, 0]


def kernel(pos_src, pos_dst, neg_src, neg_dst, node_embeddings):
    src = jnp.concatenate([pos_src, neg_src]).astype(jnp.int32)
    dst = jnp.concatenate([pos_dst, neg_dst]).astype(jnp.int32)
    emb_bf = node_embeddings.astype(jnp.bfloat16)
    emb_i32 = jax.lax.bitcast_convert_type(
        emb_bf.reshape(N_NODES, D // 2, 2), jnp.int32)
    scores = _scores_sc(src, dst, emb_i32)
    return _loss_tc(scores)
